# Initial kernel scaffold; baseline (speedup 1.0000x reference)
import jax
import jax.numpy as jnp
from jax.experimental import pallas as pl


def kernel(feats, xyz0, sxyz0, sfeats0, xyz1, sxyz1, sfeats1, xyz2, sxyz2, sfeats2, xyz3, sxyz3, sfeats3, u0_ln1_g, u0_ln1_b, u0_w1, u0_b1, u0_ln2_g, u0_ln2_b, u0_w2, u0_b2, u1_ln1_g, u1_ln1_b, u1_w1, u1_b1, u1_ln2_g, u1_ln2_b, u1_w2, u1_b2, u2_ln1_g, u2_ln1_b, u2_w1, u2_b1, u2_ln2_g, u2_ln2_b, u2_w2, u2_b2, u3_ln1_g, u3_ln1_b, u3_w1, u3_b1, u3_ln2_g, u3_ln2_b, u3_w2, u3_b2, conf_w1, conf_b1, conf_bn_g, conf_bn_b, conf_w2, conf_b2):
    raise NotImplementedError("write your pallas kernel here")


# speedup vs baseline: 4.6289x; 4.6289x over previous
import functools

import jax
import jax.numpy as jnp
from jax.experimental import pallas as pl
from jax.experimental.pallas import tpu as pltpu

_F32 = jnp.float32


def _ln(x, g, b):
    mu = jnp.mean(x, axis=-1, keepdims=True)
    var = jnp.mean((x - mu) ** 2, axis=-1, keepdims=True)
    return (x - mu) * jax.lax.rsqrt(var + 1e-5) * g + b


def _stage_kernel(xyz_ref, sxyz_ref, sfeats_ref, fprev_ref,
                  g1_ref, b1_ref, w1_ref, bb1_ref,
                  g2_ref, bt2_ref, w2_ref, bb2_ref,
                  out_ref, f2_buf):
    i = pl.program_id(0)

    @pl.when(i == 0)
    def _():
        fp = fprev_ref[...]
        f2 = _ln(fp, g2_ref[...], bt2_ref[...])
        f2_buf[...] = (jnp.dot(f2, w2_ref[...], preferred_element_type=_F32)
                       + bb2_ref[...])

    sf = sfeats_ref[...]
    a = (jnp.dot(_ln(sf, g1_ref[...], b1_ref[...]), w1_ref[...],
                 preferred_element_type=_F32) + bb1_ref[...])

    q = sxyz_ref[...]
    xt = xyz_ref[...]
    d2 = ((q[:, 0:1] - xt[0:1, :]) ** 2
          + (q[:, 1:2] - xt[1:2, :]) ** 2
          + (q[:, 2:3] - xt[2:3, :]) ** 2)

    mpad = d2.shape[1]
    iota = jax.lax.broadcasted_iota(jnp.int32, d2.shape, 1)
    big = _F32(1e30)

    def pick(d):
        m = jnp.min(d, axis=-1, keepdims=True)
        idx = jnp.min(jnp.where(d == m, iota, mpad), axis=-1, keepdims=True)
        return m, idx

    m1, i1 = pick(d2)
    d2b = jnp.where(iota == i1, big, d2)
    m2, i2 = pick(d2b)
    d2c = jnp.where(iota == i2, big, d2b)
    m3, i3 = pick(d2c)

    def wgt(m):
        return 1.0 / (jnp.sqrt(jnp.maximum(m, 1e-10)) + 1e-8)

    wa, wb, wc = wgt(m1), wgt(m2), wgt(m3)
    ws = wa + wb + wc
    zero = _F32(0.0)
    onehot = (jnp.where(iota == i1, wa / ws, zero)
              + jnp.where(iota == i2, wb / ws, zero)
              + jnp.where(iota == i3, wc / ws, zero))
    interp = jax.lax.dot_general(
        onehot, f2_buf[...], (((1,), (0,)), ((), ())),
        precision=jax.lax.Precision.HIGHEST, preferred_element_type=_F32)
    out_ref[...] = a + interp


def _stage(fprev_pad, xyz, sxyz, sfeats,
           g1, b1, w1, bb1, g2, bt2, w2, bb2,
           mpad, spad, tile):
    m = xyz.shape[0]
    s = sxyz.shape[0]
    ci = fprev_pad.shape[1]
    co = sfeats.shape[1]
    xyz_t = jnp.pad(xyz, ((0, mpad - m), (0, 5)), constant_values=1e6).T
    sxyz_p = jnp.pad(sxyz, ((0, spad - s), (0, 5)))
    sfeats_p = jnp.pad(sfeats, ((0, spad - s), (0, 0)))
    vec = lambda v: v.reshape(1, -1)
    grid = spad // tile
    full = lambda i: (0, 0)
    tiled = lambda i: (i, 0)
    return pl.pallas_call(
        _stage_kernel,
        grid=(grid,),
        in_specs=[
            pl.BlockSpec((8, mpad), full),
            pl.BlockSpec((tile, 8), tiled),
            pl.BlockSpec((tile, co), tiled),
            pl.BlockSpec((mpad, ci), full),
            pl.BlockSpec((1, co), full),
            pl.BlockSpec((1, co), full),
            pl.BlockSpec((co, co), full),
            pl.BlockSpec((1, co), full),
            pl.BlockSpec((1, ci), full),
            pl.BlockSpec((1, ci), full),
            pl.BlockSpec((ci, co), full),
            pl.BlockSpec((1, co), full),
        ],
        out_specs=pl.BlockSpec((tile, co), tiled),
        out_shape=jax.ShapeDtypeStruct((spad, co), _F32),
        scratch_shapes=[pltpu.VMEM((mpad, co), _F32)],
    )(xyz_t, sxyz_p, sfeats_p, fprev_pad,
      vec(g1), vec(b1), w1, vec(bb1), vec(g2), vec(bt2), w2, vec(bb2))


def _head_kernel(f_ref, w1_ref, b1_ref, g_ref, bb_ref, w2_ref, b2_ref,
                 out_ref, *, n):
    f = f_ref[...]
    h = jnp.dot(f, w1_ref[...], preferred_element_type=_F32) + b1_ref[...]
    rows = jax.lax.broadcasted_iota(jnp.int32, h.shape, 0)
    mask = (rows < n).astype(_F32)
    inv = _F32(1.0 / n)
    mu = jnp.sum(h * mask, axis=0, keepdims=True) * inv
    var = jnp.sum(((h - mu) ** 2) * mask, axis=0, keepdims=True) * inv
    hn = (h - mu) * jax.lax.rsqrt(var + 1e-5) * g_ref[...] + bb_ref[...]
    hn = jnp.maximum(hn, 0.0)
    out_ref[...] = (jnp.dot(hn, w2_ref[...], preferred_element_type=_F32)
                    + b2_ref[...])


def kernel(feats, xyz0, sxyz0, sfeats0, xyz1, sxyz1, sfeats1,
           xyz2, sxyz2, sfeats2, xyz3, sxyz3, sfeats3,
           u0_ln1_g, u0_ln1_b, u0_w1, u0_b1, u0_ln2_g, u0_ln2_b, u0_w2, u0_b2,
           u1_ln1_g, u1_ln1_b, u1_w1, u1_b1, u1_ln2_g, u1_ln2_b, u1_w2, u1_b2,
           u2_ln1_g, u2_ln1_b, u2_w1, u2_b1, u2_ln2_g, u2_ln2_b, u2_w2, u2_b2,
           u3_ln1_g, u3_ln1_b, u3_w1, u3_b1, u3_ln2_g, u3_ln2_b, u3_w2, u3_b2,
           conf_w1, conf_b1, conf_bn_g, conf_bn_b, conf_w2, conf_b2):
    f0 = jnp.pad(feats, ((0, 128 - 39), (0, 0)))
    f1 = _stage(f0, xyz0, sxyz0, sfeats0,
                u0_ln1_g, u0_ln1_b, u0_w1, u0_b1,
                u0_ln2_g, u0_ln2_b, u0_w2, u0_b2,
                mpad=128, spad=160, tile=160)
    f2 = _stage(f1, xyz1, sxyz1, sfeats1,
                u1_ln1_g, u1_ln1_b, u1_w1, u1_b1,
                u1_ln2_g, u1_ln2_b, u1_w2, u1_b2,
                mpad=160, spad=640, tile=640)
    f3 = _stage(f2, xyz2, sxyz2, sfeats2,
                u2_ln1_g, u2_ln1_b, u2_w1, u2_b1,
                u2_ln2_g, u2_ln2_b, u2_w2, u2_b2,
                mpad=640, spad=2560, tile=512)
    f4 = _stage(f3, xyz3, sxyz3, sfeats3,
                u3_ln1_g, u3_ln1_b, u3_w1, u3_b1,
                u3_ln2_g, u3_ln2_b, u3_w2, u3_b2,
                mpad=2560, spad=10240, tile=256)
    n = sxyz3.shape[0]
    npad = f4.shape[0]
    vec = lambda v: v.reshape(1, -1)
    conf = pl.pallas_call(
        functools.partial(_head_kernel, n=n),
        out_shape=jax.ShapeDtypeStruct((npad, 1), _F32),
    )(f4, conf_w1, vec(conf_b1), vec(conf_bn_g), vec(conf_bn_b),
      conf_w2, vec(conf_b2))
    return conf[:n, :]


# speedup vs baseline: 8.1240x; 1.7551x over previous
import functools

import jax
import jax.numpy as jnp
from jax.experimental import pallas as pl
from jax.experimental.pallas import tpu as pltpu

_F32 = jnp.float32


def _ln(x, g, b):
    mu = jnp.mean(x, axis=-1, keepdims=True)
    var = jnp.mean((x - mu) ** 2, axis=-1, keepdims=True)
    return (x - mu) * jax.lax.rsqrt(var + 1e-5) * g + b


def _stage_kernel(xyz_ref, sxyz_ref, sfeats_ref, fprev_ref,
                  g1_ref, b1_ref, w1_ref, bb1_ref,
                  g2_ref, bt2_ref, w2_ref, bb2_ref,
                  out_ref, f2_buf):
    i = pl.program_id(0)

    @pl.when(i == 0)
    def _():
        fp = fprev_ref[...]
        f2 = _ln(fp, g2_ref[...], bt2_ref[...])
        f2_buf[...] = (jnp.dot(f2, w2_ref[...], preferred_element_type=_F32)
                       + bb2_ref[...])

    sf = sfeats_ref[...]
    a = (jnp.dot(_ln(sf, g1_ref[...], b1_ref[...]), w1_ref[...],
                 preferred_element_type=_F32) + bb1_ref[...])

    q = sxyz_ref[...]
    xt = xyz_ref[...]
    qx, qy, qz = q[:, 0:1], q[:, 1:2], q[:, 2:3]
    xx, xy, xz = xt[0:1, :], xt[1:2, :], xt[2:3, :]
    sq = qx * qx + qy * qy + qz * qz
    r = xx * xx + xy * xy + xz * xz
    d2 = (((r + (-2.0 * qx) * xx) + (-2.0 * qy) * xy)
          + (-2.0 * qz) * xz) + sq

    big = _F32(1e30)
    v1 = jnp.min(d2, axis=-1, keepdims=True)
    eq1 = d2 == v1
    dm = jnp.where(eq1, big, d2)
    v2 = jnp.min(dm, axis=-1, keepdims=True)
    eq2 = dm == v2
    dm2 = jnp.where(eq2, big, dm)
    v3 = jnp.min(dm2, axis=-1, keepdims=True)
    eq3 = dm2 == v3

    def wgt(m):
        return 1.0 / (jnp.sqrt(jnp.maximum(m, 1e-10)) + 1e-8)

    wa, wb, wc = wgt(v1), wgt(v2), wgt(v3)
    inv = 1.0 / (wa + wb + wc)
    zero = _F32(0.0)
    onehot = (jnp.where(eq1, wa * inv, zero)
              + jnp.where(eq2, wb * inv, zero)
              + jnp.where(eq3, wc * inv, zero))
    interp = jnp.dot(onehot, f2_buf[...], preferred_element_type=_F32)
    out_ref[...] = a + interp


def _stage(fprev_pad, xyz, sxyz, sfeats,
           g1, b1, w1, bb1, g2, bt2, w2, bb2,
           mpad, spad, tile):
    m = xyz.shape[0]
    s = sxyz.shape[0]
    ci = fprev_pad.shape[1]
    co = sfeats.shape[1]
    xyz_t = jnp.pad(xyz, ((0, mpad - m), (0, 5)), constant_values=1e6).T
    sxyz_p = jnp.pad(sxyz, ((0, spad - s), (0, 5)))
    sfeats_p = jnp.pad(sfeats, ((0, spad - s), (0, 0)))
    vec = lambda v: v.reshape(1, -1)
    grid = spad // tile
    full = lambda i: (0, 0)
    tiled = lambda i: (i, 0)
    return pl.pallas_call(
        _stage_kernel,
        grid=(grid,),
        in_specs=[
            pl.BlockSpec((8, mpad), full),
            pl.BlockSpec((tile, 8), tiled),
            pl.BlockSpec((tile, co), tiled),
            pl.BlockSpec((mpad, ci), full),
            pl.BlockSpec((1, co), full),
            pl.BlockSpec((1, co), full),
            pl.BlockSpec((co, co), full),
            pl.BlockSpec((1, co), full),
            pl.BlockSpec((1, ci), full),
            pl.BlockSpec((1, ci), full),
            pl.BlockSpec((ci, co), full),
            pl.BlockSpec((1, co), full),
        ],
        out_specs=pl.BlockSpec((tile, co), tiled),
        out_shape=jax.ShapeDtypeStruct((spad, co), _F32),
        scratch_shapes=[pltpu.VMEM((mpad, co), _F32)],
    )(xyz_t, sxyz_p, sfeats_p, fprev_pad,
      vec(g1), vec(b1), w1, vec(bb1), vec(g2), vec(bt2), w2, vec(bb2))


def _head_kernel(f_ref, w1_ref, b1_ref, g_ref, bb_ref, w2_ref, b2_ref,
                 out_ref, *, n):
    f = f_ref[...]
    h = jnp.dot(f, w1_ref[...], preferred_element_type=_F32) + b1_ref[...]
    rows = jax.lax.broadcasted_iota(jnp.int32, h.shape, 0)
    mask = (rows < n).astype(_F32)
    inv = _F32(1.0 / n)
    mu = jnp.sum(h * mask, axis=0, keepdims=True) * inv
    var = jnp.sum(((h - mu) ** 2) * mask, axis=0, keepdims=True) * inv
    hn = (h - mu) * jax.lax.rsqrt(var + 1e-5) * g_ref[...] + bb_ref[...]
    hn = jnp.maximum(hn, 0.0)
    out_ref[...] = (jnp.dot(hn, w2_ref[...], preferred_element_type=_F32)
                    + b2_ref[...])


def kernel(feats, xyz0, sxyz0, sfeats0, xyz1, sxyz1, sfeats1,
           xyz2, sxyz2, sfeats2, xyz3, sxyz3, sfeats3,
           u0_ln1_g, u0_ln1_b, u0_w1, u0_b1, u0_ln2_g, u0_ln2_b, u0_w2, u0_b2,
           u1_ln1_g, u1_ln1_b, u1_w1, u1_b1, u1_ln2_g, u1_ln2_b, u1_w2, u1_b2,
           u2_ln1_g, u2_ln1_b, u2_w1, u2_b1, u2_ln2_g, u2_ln2_b, u2_w2, u2_b2,
           u3_ln1_g, u3_ln1_b, u3_w1, u3_b1, u3_ln2_g, u3_ln2_b, u3_w2, u3_b2,
           conf_w1, conf_b1, conf_bn_g, conf_bn_b, conf_w2, conf_b2):
    f0 = jnp.pad(feats, ((0, 128 - 39), (0, 0)))
    f1 = _stage(f0, xyz0, sxyz0, sfeats0,
                u0_ln1_g, u0_ln1_b, u0_w1, u0_b1,
                u0_ln2_g, u0_ln2_b, u0_w2, u0_b2,
                mpad=128, spad=160, tile=160)
    f2 = _stage(f1, xyz1, sxyz1, sfeats1,
                u1_ln1_g, u1_ln1_b, u1_w1, u1_b1,
                u1_ln2_g, u1_ln2_b, u1_w2, u1_b2,
                mpad=160, spad=640, tile=640)
    f3 = _stage(f2, xyz2, sxyz2, sfeats2,
                u2_ln1_g, u2_ln1_b, u2_w1, u2_b1,
                u2_ln2_g, u2_ln2_b, u2_w2, u2_b2,
                mpad=640, spad=2560, tile=512)
    f4 = _stage(f3, xyz3, sxyz3, sfeats3,
                u3_ln1_g, u3_ln1_b, u3_w1, u3_b1,
                u3_ln2_g, u3_ln2_b, u3_w2, u3_b2,
                mpad=2560, spad=10240, tile=256)
    n = sxyz3.shape[0]
    npad = f4.shape[0]
    vec = lambda v: v.reshape(1, -1)
    conf = pl.pallas_call(
        functools.partial(_head_kernel, n=n),
        out_shape=jax.ShapeDtypeStruct((npad, 1), _F32),
    )(f4, conf_w1, vec(conf_b1), vec(conf_bn_g), vec(conf_bn_b),
      conf_w2, vec(conf_b2))
    return conf[:n, :]
